# baseline (speedup 1.0000x reference)
import functools

import jax
import jax.numpy as jnp
from jax import lax
from jax.experimental import pallas as pl
from jax.experimental.pallas import tpu as pltpu

_B, _K, _P = 32, 16, 1024
_SLOT_DIM, _DEC_DIM, _OUT_DIM, _TOP_K = 128, 128, 96, 4
_BB = 4


def _tc_body(slots_ref, masks_ref, w_in_ref, b_in_ref, pos_ref, w_dec_ref,
             b_dec_ref, recon_ref, masks_all_ref, vt_scr):
    g = pl.program_id(0)

    @pl.when(g == 0)
    def _():
        vt = lax.dot_general(w_dec_ref[...], pos_ref[...],
                             (((1,), (1,)), ((), ())),
                             preferred_element_type=jnp.float32)
        vt_scr[...] = vt + b_dec_ref[...]

    s2 = slots_ref[...].reshape(_BB * _K, _SLOT_DIM)
    s = lax.dot_general(s2, w_in_ref[...], (((1,), (1,)), ((), ())),
                        preferred_element_type=jnp.float32) + b_in_ref[...]
    u = lax.dot_general(s, w_dec_ref[...], (((1,), (1,)), ((), ())),
                        preferred_element_type=jnp.float32)
    ua = u[:, _OUT_DIM:_OUT_DIM + 1].reshape(_BB, _K, 1)

    m = masks_ref[...]
    kio = lax.broadcasted_iota(jnp.int32, (_BB, _K, _P), 1)
    neg = jnp.float32(-jnp.inf)
    selmask = jnp.zeros((_BB, _K, _P), dtype=jnp.bool_)
    work = m
    for _ in range(_TOP_K):
        colmax = jnp.max(work, axis=1, keepdims=True)
        ismax = work == colmax
        first = jnp.min(jnp.where(ismax, kio, _K), axis=1, keepdims=True)
        fm = kio == first
        selmask = jnp.logical_or(selmask, fm)
        work = jnp.where(fm, neg, work)

    uab = jnp.broadcast_to(ua, (_BB, _K, _P))
    rowmax = jnp.max(jnp.where(selmask, uab, neg), axis=1, keepdims=True)
    e = jnp.where(selmask, jnp.exp(uab - rowmax), 0.0)
    wm = e / jnp.sum(e, axis=1, keepdims=True)
    masks_all_ref[...] = wm
    vt96 = vt_scr[:_OUT_DIM, :]
    for bb in range(_BB):
        u96 = u[bb * _K:(bb + 1) * _K, :_OUT_DIM]
        recon_t = lax.dot_general(u96, wm[bb], (((0,), (0,)), ((), ())),
                                  preferred_element_type=jnp.float32)
        recon_ref[bb] = recon_t + vt96


@jax.jit
def kernel(slots, masks, W_in, b_in, pos_embed, W_dec, b_dec):
    w_dec_p = jnp.zeros((_DEC_DIM, _DEC_DIM), jnp.float32).at[:_OUT_DIM + 1].set(W_dec)
    b_dec_p = jnp.zeros((_DEC_DIM, 1), jnp.float32).at[:_OUT_DIM + 1, 0].set(b_dec)
    pos2d = pos_embed.reshape(_P, _DEC_DIM)
    b_in2d = b_in.reshape(1, _DEC_DIM)

    grid = (_B // _BB,)
    recon_t, masks_all = pl.pallas_call(
        _tc_body,
        grid=grid,
        in_specs=[
            pl.BlockSpec((_BB, _K, _SLOT_DIM), lambda g: (g, 0, 0)),
            pl.BlockSpec((_BB, _K, _P), lambda g: (g, 0, 0)),
            pl.BlockSpec((_DEC_DIM, _SLOT_DIM), lambda g: (0, 0)),
            pl.BlockSpec((1, _DEC_DIM), lambda g: (0, 0)),
            pl.BlockSpec((_P, _DEC_DIM), lambda g: (0, 0)),
            pl.BlockSpec((_DEC_DIM, _DEC_DIM), lambda g: (0, 0)),
            pl.BlockSpec((_DEC_DIM, 1), lambda g: (0, 0)),
        ],
        out_specs=[
            pl.BlockSpec((_BB, _OUT_DIM, _P), lambda g: (g, 0, 0)),
            pl.BlockSpec((_BB, _K, _P), lambda g: (g, 0, 0)),
        ],
        out_shape=[
            jax.ShapeDtypeStruct((_B, _OUT_DIM, _P), jnp.float32),
            jax.ShapeDtypeStruct((_B, _K, _P), jnp.float32),
        ],
        scratch_shapes=[pltpu.VMEM((_DEC_DIM, _P), jnp.float32)],
        compiler_params=pltpu.CompilerParams(
            dimension_semantics=("arbitrary",)),
    )(slots, masks, W_in, b_in2d, pos2d, w_dec_p, b_dec_p)
    return jnp.swapaxes(recon_t, 1, 2), masks_all


# speedup vs baseline: 1.1504x; 1.1504x over previous
import functools

import jax
import jax.numpy as jnp
from jax import lax
from jax.experimental import pallas as pl
from jax.experimental.pallas import tpu as pltpu

_B, _K, _P = 32, 16, 1024
_SLOT_DIM, _DEC_DIM, _OUT_DIM, _TOP_K = 128, 128, 96, 4
_BB = 16


def _tc_body(slots_ref, masks_ref, w_in_ref, b_in_ref, pos_ref, w_dec_ref,
             b_dec_ref, recon_ref, masks_all_ref, vt_scr):
    g = pl.program_id(0)

    @pl.when(g == 0)
    def _():
        vt = lax.dot_general(w_dec_ref[...], pos_ref[...],
                             (((1,), (1,)), ((), ())),
                             preferred_element_type=jnp.float32)
        vt_scr[...] = vt + b_dec_ref[...]

    s2 = slots_ref[...].reshape(_BB * _K, _SLOT_DIM)
    s = lax.dot_general(s2, w_in_ref[...], (((1,), (1,)), ((), ())),
                        preferred_element_type=jnp.float32) + b_in_ref[...]
    u = lax.dot_general(s, w_dec_ref[...], (((1,), (1,)), ((), ())),
                        preferred_element_type=jnp.float32)
    ua = u[:, _OUT_DIM:_OUT_DIM + 1].reshape(_BB, _K, 1)

    m = masks_ref[...]
    kio = lax.broadcasted_iota(jnp.int32, (_BB, _K, _P), 1)
    neg = jnp.float32(-jnp.inf)
    selmask = jnp.zeros((_BB, _K, _P), dtype=jnp.bool_)
    work = m
    for _ in range(_TOP_K):
        colmax = jnp.max(work, axis=1, keepdims=True)
        ismax = work == colmax
        first = jnp.min(jnp.where(ismax, kio, _K), axis=1, keepdims=True)
        fm = kio == first
        selmask = jnp.logical_or(selmask, fm)
        work = jnp.where(fm, neg, work)

    uab = jnp.broadcast_to(ua, (_BB, _K, _P))
    rowmax = jnp.max(jnp.where(selmask, uab, neg), axis=1, keepdims=True)
    e = jnp.where(selmask, jnp.exp(uab - rowmax), 0.0)
    wm = e / jnp.sum(e, axis=1, keepdims=True)
    masks_all_ref[...] = wm
    vt96 = vt_scr[:_OUT_DIM, :]
    for bb in range(_BB):
        u96 = u[bb * _K:(bb + 1) * _K, :_OUT_DIM]
        recon_t = lax.dot_general(u96, wm[bb], (((0,), (0,)), ((), ())),
                                  preferred_element_type=jnp.float32)
        recon_ref[bb] = recon_t + vt96


@jax.jit
def kernel(slots, masks, W_in, b_in, pos_embed, W_dec, b_dec):
    w_dec_p = jnp.zeros((_DEC_DIM, _DEC_DIM), jnp.float32).at[:_OUT_DIM + 1].set(W_dec)
    b_dec_p = jnp.zeros((_DEC_DIM, 1), jnp.float32).at[:_OUT_DIM + 1, 0].set(b_dec)
    pos2d = pos_embed.reshape(_P, _DEC_DIM)
    b_in2d = b_in.reshape(1, _DEC_DIM)

    grid = (_B // _BB,)
    recon_t, masks_all = pl.pallas_call(
        _tc_body,
        grid=grid,
        in_specs=[
            pl.BlockSpec((_BB, _K, _SLOT_DIM), lambda g: (g, 0, 0)),
            pl.BlockSpec((_BB, _K, _P), lambda g: (g, 0, 0)),
            pl.BlockSpec((_DEC_DIM, _SLOT_DIM), lambda g: (0, 0)),
            pl.BlockSpec((1, _DEC_DIM), lambda g: (0, 0)),
            pl.BlockSpec((_P, _DEC_DIM), lambda g: (0, 0)),
            pl.BlockSpec((_DEC_DIM, _DEC_DIM), lambda g: (0, 0)),
            pl.BlockSpec((_DEC_DIM, 1), lambda g: (0, 0)),
        ],
        out_specs=[
            pl.BlockSpec((_BB, _OUT_DIM, _P), lambda g: (g, 0, 0)),
            pl.BlockSpec((_BB, _K, _P), lambda g: (g, 0, 0)),
        ],
        out_shape=[
            jax.ShapeDtypeStruct((_B, _OUT_DIM, _P), jnp.float32),
            jax.ShapeDtypeStruct((_B, _K, _P), jnp.float32),
        ],
        scratch_shapes=[pltpu.VMEM((_DEC_DIM, _P), jnp.float32)],
        compiler_params=pltpu.CompilerParams(
            dimension_semantics=("arbitrary",)),
    )(slots, masks, W_in, b_in2d, pos2d, w_dec_p, b_dec_p)
    return jnp.swapaxes(recon_t, 1, 2), masks_all
